# Initial kernel scaffold; baseline (speedup 1.0000x reference)
import functools

import numpy as np

import jax
import jax.numpy as jnp
from jax import lax
from jax.experimental import pallas as pl
from jax.experimental.pallas import tpu as pltpu
from jax.experimental.pallas import tpu_sc as plsc

_NODE_NUM = [1024, 768, 512, 1024, 896, 640, 1024, 1024]
_B = len(_NODE_NUM)
_D = 128
_MAXN = 1024
_MAXE = 3072
_MAXLEN = _MAXN + _MAXE + 1
_TOK = 4104

_NSEC = _B * _MAXN
_ESEC = _B * _MAXE
_GTOT = _NSEC + 2 * _ESEC

_NW = 32
_PW = _GTOT // _NW
_CH = 128
_NCH = _PW // _CH

_TOTAL_NODES = sum(_NODE_NUM)


def _sc_gather(k_idx, subnodes, orf0):
    mesh = plsc.VectorSubcoreMesh(core_axis_name="c", subcore_axis_name="s")

    @functools.partial(
        pl.kernel,
        out_type=jax.ShapeDtypeStruct((_GTOT, _D), jnp.float32),
        mesh=mesh,
        scratch_types=[
            pltpu.VMEM((_TOTAL_NODES,), jnp.int32),
            pltpu.VMEM((_PW,), jnp.int32),
            pltpu.VMEM((_PW,), jnp.int32),
            pltpu.VMEM((2, _CH, _D), jnp.float32),
            pltpu.SemaphoreType.DMA,
        ],
    )
    def run(k_hbm, sub_hbm, orf_hbm, out_hbm, sub_v, kv, fv, rows, sem):
        wid = lax.axis_index("s") * 2 + lax.axis_index("c")
        base = wid * _PW
        pltpu.sync_copy(sub_hbm, sub_v)
        pltpu.sync_copy(k_hbm.at[pl.ds(base, _PW)], kv)

        def compose(i, carry):
            idx = kv[pl.ds(i * 16, 16)]
            fv[pl.ds(i * 16, 16)] = plsc.load_gather(sub_v, [idx])
            return carry

        lax.fori_loop(0, _PW // 16, compose, 0)

        copies = [None, None]
        for c in range(_NCH):
            buf = c % 2
            copies[buf] = pltpu.async_copy(
                orf_hbm.at[fv.at[pl.ds(c * _CH, _CH)]], rows.at[buf], sem)
            if c > 0:
                copies[1 - buf].wait()
                pltpu.sync_copy(rows.at[1 - buf],
                                out_hbm.at[pl.ds(base + (c - 1) * _CH, _CH)])
        copies[(_NCH - 1) % 2].wait()
        pltpu.sync_copy(rows.at[(_NCH - 1) % 2],
                        out_hbm.at[pl.ds(base + (_NCH - 1) * _CH, _CH)])

    return run(k_idx, subnodes, orf0)


def _tc_body(num_ref, node_ref, edge_ref, gn_ref, gs_ref, gd_ref,
             es_ref, ed_ref, a1_ref, a2_ref, a12_ref, ow_ref,
             outf_ref, ps_ref, pd_ref):
    g = pl.program_id(0)
    n = num_ref[g]
    n3 = n * 3
    ow0 = ow_ref[0:1, :]
    ow1 = ow_ref[1:2, :]

    cn = (node_ref[...]
          + jnp.dot(gn_ref[...], a12_ref[...],
                    preferred_element_type=jnp.float32,
                    precision=lax.Precision.HIGHEST)
          + ow1)

    es = es_ref[...]
    ed = ed_ref[...]
    eqf = (es == ed).astype(jnp.float32)
    owe = ow0 + eqf * (ow1 - ow0)
    ce = (edge_ref[...]
          + jnp.dot(gs_ref[...], a1_ref[...],
                    preferred_element_type=jnp.float32,
                    precision=lax.Precision.HIGHEST)
          + jnp.dot(gd_ref[...], a2_ref[...],
                    preferred_element_type=jnp.float32,
                    precision=lax.Precision.HIGHEST)
          + owe)
    j = lax.broadcasted_iota(jnp.int32, (_MAXE, 1), 0)
    emask = j < n3
    ce = jnp.where(emask, ce, 0.0)

    outf_ref[...] = jnp.zeros((_TOK, _D), jnp.float32)
    outf_ref[0:_MAXN, :] = cn
    outf_ref[pl.ds(n, _MAXE), :] = ce

    t = lax.broadcasted_iota(jnp.int32, (_TOK, 1), 0)
    base = jnp.where(t < n, t, 0)
    zi = jnp.zeros((), jnp.int32)
    ps_ref[...] = base
    pd_ref[...] = base
    ps_ref[pl.ds(n, _MAXE), :] = jnp.where(emask, es, zi)
    pd_ref[pl.ds(n, _MAXE), :] = jnp.where(emask, ed, zi)


def _tc_call(num, node_p, edge_p, gn, gs, gd, es_col, ed_col, a1, a2, a12, ow):
    def per_graph(*blk):
        return pl.BlockSpec(blk, lambda g: (g,) + (0,) * (len(blk) - 1))

    def shared(*blk):
        return pl.BlockSpec(blk, lambda g: (0,) * len(blk))

    return pl.pallas_call(
        _tc_body,
        grid=(_B,),
        in_specs=[
            pl.BlockSpec(memory_space=pltpu.SMEM),
            per_graph(None, _MAXN, _D),
            per_graph(None, _MAXE, _D),
            per_graph(None, _MAXN, _D),
            per_graph(None, _MAXE, _D),
            per_graph(None, _MAXE, _D),
            per_graph(None, _MAXE, 1),
            per_graph(None, _MAXE, 1),
            shared(_D, _D),
            shared(_D, _D),
            shared(_D, _D),
            shared(2, _D),
        ],
        out_specs=[
            per_graph(None, _TOK, _D),
            per_graph(None, _TOK, 1),
            per_graph(None, _TOK, 1),
        ],
        out_shape=[
            jax.ShapeDtypeStruct((_B, _TOK, _D), jnp.float32),
            jax.ShapeDtypeStruct((_B, _TOK, 1), jnp.int32),
            jax.ShapeDtypeStruct((_B, _TOK, 1), jnp.int32),
        ],
        compiler_params=pltpu.CompilerParams(
            dimension_semantics=("arbitrary",)),
    )(num, node_p, edge_p, gn, gs, gd, es_col, ed_col, a1, a2, a12, ow)


def _static_setup():
    node_num = np.asarray(_NODE_NUM, np.int32)
    noff = np.concatenate([[0], np.cumsum(node_num)]).astype(np.int64)
    eoff = 3 * noff
    k1 = np.zeros((_B, _MAXN), np.int32)
    for g, n in enumerate(_NODE_NUM):
        k1[g, :n] = noff[g] + np.arange(n)
    tok = np.arange(_MAXLEN, dtype=np.int32)[None, :]
    nn = node_num[:, None]
    seq = 4 * nn
    padded_node_mask = tok < nn
    padded_edge_mask = (tok >= nn) & (tok < seq)
    padding_mask = tok >= seq
    return noff, eoff, k1.reshape(-1), padding_mask, padded_node_mask, padded_edge_mask


_NOFF, _EOFF, _K1, _PAD_MASK, _NODE_MASK, _EDGE_MASK = _static_setup()


def kernel(node_data, edge_data, edge_index, indices_subnodes,
           order_weight, orf_w, orf):
    orf0 = orf[0]
    k2, k3, es_cols, ed_cols, node_ps, edge_ps = [], [], [], [], [], []
    for g, n in enumerate(_NODE_NUM):
        e = 3 * n
        src = edge_index[0, _EOFF[g]:_EOFF[g] + e]
        dst = edge_index[1, _EOFF[g]:_EOFF[g] + e]
        k2.append(jnp.pad(src + np.int32(_NOFF[g]), (0, _MAXE - e)))
        k3.append(jnp.pad(dst + np.int32(_NOFF[g]), (0, _MAXE - e)))
        es_cols.append(jnp.pad(src, (0, _MAXE - e)))
        ed_cols.append(jnp.pad(dst, (0, _MAXE - e)))
        node_ps.append(jnp.pad(node_data[_NOFF[g]:_NOFF[g] + n],
                               ((0, _MAXN - n), (0, 0))))
        edge_ps.append(jnp.pad(edge_data[_EOFF[g]:_EOFF[g] + e],
                               ((0, _MAXE - e), (0, 0))))
    k_idx = jnp.concatenate([jnp.asarray(_K1)] + k2 + k3).astype(jnp.int32)
    node_p = jnp.stack(node_ps)
    edge_p = jnp.stack(edge_ps)
    es_col = jnp.stack(es_cols).astype(jnp.int32)[..., None]
    ed_col = jnp.stack(ed_cols).astype(jnp.int32)[..., None]

    gall = _sc_gather(k_idx, indices_subnodes, orf0)
    gn = gall[:_NSEC].reshape(_B, _MAXN, _D)
    gs = gall[_NSEC:_NSEC + _ESEC].reshape(_B, _MAXE, _D)
    gd = gall[_NSEC + _ESEC:].reshape(_B, _MAXE, _D)

    a1 = orf_w[:, :_D].T
    a2 = orf_w[:, _D:].T
    a12 = a1 + a2
    num = jnp.asarray(np.asarray(_NODE_NUM, np.int32))

    outf, ps, pd = _tc_call(num, node_p, edge_p, gn, gs, gd,
                            es_col, ed_col, a1, a2, a12, order_weight)

    padded_feature = outf[:, :_MAXLEN, :]
    padded_index = jnp.stack([ps[:, :_MAXLEN, 0], pd[:, :_MAXLEN, 0]], axis=-1)
    return (padded_feature,
            jnp.asarray(_PAD_MASK),
            padded_index,
            jnp.asarray(_NODE_MASK),
            jnp.asarray(_EDGE_MASK))


# speedup vs baseline: 2.2327x; 2.2327x over previous
import functools

import numpy as np

import jax
import jax.numpy as jnp
from jax import lax
from jax.experimental import pallas as pl
from jax.experimental.pallas import tpu as pltpu
from jax.experimental.pallas import tpu_sc as plsc

_NODE_NUM = [1024, 768, 512, 1024, 896, 640, 1024, 1024]
_B = len(_NODE_NUM)
_D = 128
_MAXN = 1024
_MAXE = 3072
_MAXLEN = _MAXN + _MAXE + 1
_TOK = 4104

_NSEC = _B * _MAXN
_ESEC = _B * _MAXE
_GTOT = _NSEC + 2 * _ESEC

_NW = 32
_PW = _GTOT // _NW
_CH = 128
_NCH = _PW // _CH

_TOTAL_NODES = sum(_NODE_NUM)


def _sc_gather(k_idx, subnodes, orf0):
    mesh = plsc.VectorSubcoreMesh(core_axis_name="c", subcore_axis_name="s")

    @functools.partial(
        pl.kernel,
        out_type=jax.ShapeDtypeStruct((_GTOT, _D), jnp.float32),
        mesh=mesh,
        scratch_types=[
            pltpu.VMEM((_TOTAL_NODES,), jnp.int32),
            pltpu.VMEM((_PW,), jnp.int32),
            pltpu.VMEM((_PW,), jnp.int32),
            pltpu.VMEM((2, _CH, _D), jnp.float32),
            pltpu.SemaphoreType.DMA,
        ],
        compiler_params=pltpu.CompilerParams(needs_layout_passes=False),
    )
    def run(k_hbm, sub_hbm, orf_hbm, out_hbm, sub_v, kv, fv, rows, sem):
        wid = lax.axis_index("s") * 2 + lax.axis_index("c")
        base = wid * _PW
        pltpu.sync_copy(sub_hbm, sub_v)
        pltpu.sync_copy(k_hbm.at[pl.ds(base, _PW)], kv)

        def compose(i, carry):
            idx = kv[pl.ds(i * 16, 16)]
            fv[pl.ds(i * 16, 16)] = plsc.load_gather(sub_v, [idx])
            return carry

        lax.fori_loop(0, _PW // 16, compose, 0)

        copies = [None, None]
        for c in range(_NCH):
            buf = c % 2
            copies[buf] = pltpu.async_copy(
                orf_hbm.at[fv.at[pl.ds(c * _CH, _CH)]], rows.at[buf], sem)
            if c > 0:
                copies[1 - buf].wait()
                pltpu.sync_copy(rows.at[1 - buf],
                                out_hbm.at[pl.ds(base + (c - 1) * _CH, _CH)])
        copies[(_NCH - 1) % 2].wait()
        pltpu.sync_copy(rows.at[(_NCH - 1) % 2],
                        out_hbm.at[pl.ds(base + (_NCH - 1) * _CH, _CH)])

    return run(k_idx, subnodes, orf0)


def _tc_body(num_ref, node_ref, edge_ref, gn_ref, gs_ref, gd_ref,
             es_ref, ed_ref, a1_ref, a2_ref, a12_ref, ow_ref,
             outf_ref, ps_ref, pd_ref):
    g = pl.program_id(0)
    n = num_ref[g]
    n3 = n * 3
    ow0 = ow_ref[0:1, :]
    ow1 = ow_ref[1:2, :]

    cn = (node_ref[...]
          + jnp.dot(gn_ref[...], a12_ref[...],
                    preferred_element_type=jnp.float32,
                    precision=lax.Precision.HIGHEST)
          + ow1)

    es = es_ref[...]
    ed = ed_ref[...]
    eqf = (es == ed).astype(jnp.float32)
    owe = ow0 + eqf * (ow1 - ow0)
    ce = (edge_ref[...]
          + jnp.dot(gs_ref[...], a1_ref[...],
                    preferred_element_type=jnp.float32,
                    precision=lax.Precision.HIGHEST)
          + jnp.dot(gd_ref[...], a2_ref[...],
                    preferred_element_type=jnp.float32,
                    precision=lax.Precision.HIGHEST)
          + owe)
    j = lax.broadcasted_iota(jnp.int32, (_MAXE, 1), 0)
    emask = j < n3
    ce = jnp.where(emask, ce, 0.0)

    outf_ref[...] = jnp.zeros((_TOK, _D), jnp.float32)
    outf_ref[0:_MAXN, :] = cn
    outf_ref[pl.ds(n, _MAXE), :] = ce

    t = lax.broadcasted_iota(jnp.int32, (_TOK, 1), 0)
    base = jnp.where(t < n, t, 0)
    zi = jnp.zeros((), jnp.int32)
    ps_ref[...] = base
    pd_ref[...] = base
    ps_ref[pl.ds(n, _MAXE), :] = jnp.where(emask, es, zi)
    pd_ref[pl.ds(n, _MAXE), :] = jnp.where(emask, ed, zi)


def _tc_call(num, node_p, edge_p, gn, gs, gd, es_col, ed_col, a1, a2, a12, ow):
    def per_graph(*blk):
        return pl.BlockSpec(blk, lambda g: (g,) + (0,) * (len(blk) - 1))

    def shared(*blk):
        return pl.BlockSpec(blk, lambda g: (0,) * len(blk))

    return pl.pallas_call(
        _tc_body,
        grid=(_B,),
        in_specs=[
            pl.BlockSpec(memory_space=pltpu.SMEM),
            per_graph(None, _MAXN, _D),
            per_graph(None, _MAXE, _D),
            per_graph(None, _MAXN, _D),
            per_graph(None, _MAXE, _D),
            per_graph(None, _MAXE, _D),
            per_graph(None, _MAXE, 1),
            per_graph(None, _MAXE, 1),
            shared(_D, _D),
            shared(_D, _D),
            shared(_D, _D),
            shared(2, _D),
        ],
        out_specs=[
            per_graph(None, _TOK, _D),
            per_graph(None, _TOK, 1),
            per_graph(None, _TOK, 1),
        ],
        out_shape=[
            jax.ShapeDtypeStruct((_B, _TOK, _D), jnp.float32),
            jax.ShapeDtypeStruct((_B, _TOK, 1), jnp.int32),
            jax.ShapeDtypeStruct((_B, _TOK, 1), jnp.int32),
        ],
        compiler_params=pltpu.CompilerParams(
            dimension_semantics=("arbitrary",)),
    )(num, node_p, edge_p, gn, gs, gd, es_col, ed_col, a1, a2, a12, ow)


def _static_setup():
    node_num = np.asarray(_NODE_NUM, np.int32)
    noff = np.concatenate([[0], np.cumsum(node_num)]).astype(np.int64)
    eoff = 3 * noff
    k1 = np.zeros((_B, _MAXN), np.int32)
    for g, n in enumerate(_NODE_NUM):
        k1[g, :n] = noff[g] + np.arange(n)
    tok = np.arange(_MAXLEN, dtype=np.int32)[None, :]
    nn = node_num[:, None]
    seq = 4 * nn
    padded_node_mask = tok < nn
    padded_edge_mask = (tok >= nn) & (tok < seq)
    padding_mask = tok >= seq
    return noff, eoff, k1.reshape(-1), padding_mask, padded_node_mask, padded_edge_mask


_NOFF, _EOFF, _K1, _PAD_MASK, _NODE_MASK, _EDGE_MASK = _static_setup()


def kernel(node_data, edge_data, edge_index, indices_subnodes,
           order_weight, orf_w, orf):
    orf0 = orf[0]
    k2, k3, es_cols, ed_cols, node_ps, edge_ps = [], [], [], [], [], []
    for g, n in enumerate(_NODE_NUM):
        e = 3 * n
        src = edge_index[0, _EOFF[g]:_EOFF[g] + e]
        dst = edge_index[1, _EOFF[g]:_EOFF[g] + e]
        k2.append(jnp.pad(src + np.int32(_NOFF[g]), (0, _MAXE - e)))
        k3.append(jnp.pad(dst + np.int32(_NOFF[g]), (0, _MAXE - e)))
        es_cols.append(jnp.pad(src, (0, _MAXE - e)))
        ed_cols.append(jnp.pad(dst, (0, _MAXE - e)))
        node_ps.append(jnp.pad(node_data[_NOFF[g]:_NOFF[g] + n],
                               ((0, _MAXN - n), (0, 0))))
        edge_ps.append(jnp.pad(edge_data[_EOFF[g]:_EOFF[g] + e],
                               ((0, _MAXE - e), (0, 0))))
    k_idx = jnp.concatenate([jnp.asarray(_K1)] + k2 + k3).astype(jnp.int32)
    node_p = jnp.stack(node_ps)
    edge_p = jnp.stack(edge_ps)
    es_col = jnp.stack(es_cols).astype(jnp.int32)[..., None]
    ed_col = jnp.stack(ed_cols).astype(jnp.int32)[..., None]

    gall = _sc_gather(k_idx, indices_subnodes, orf0)
    gn = gall[:_NSEC].reshape(_B, _MAXN, _D)
    gs = gall[_NSEC:_NSEC + _ESEC].reshape(_B, _MAXE, _D)
    gd = gall[_NSEC + _ESEC:].reshape(_B, _MAXE, _D)

    a1 = orf_w[:, :_D].T
    a2 = orf_w[:, _D:].T
    a12 = a1 + a2
    num = jnp.asarray(np.asarray(_NODE_NUM, np.int32))

    outf, ps, pd = _tc_call(num, node_p, edge_p, gn, gs, gd,
                            es_col, ed_col, a1, a2, a12, order_weight)

    padded_feature = outf[:, :_MAXLEN, :]
    padded_index = jnp.stack([ps[:, :_MAXLEN, 0], pd[:, :_MAXLEN, 0]], axis=-1)
    return (padded_feature,
            jnp.asarray(_PAD_MASK),
            padded_index,
            jnp.asarray(_NODE_MASK),
            jnp.asarray(_EDGE_MASK))


# speedup vs baseline: 2.2395x; 1.0030x over previous
import functools

import numpy as np

import jax
import jax.numpy as jnp
from jax import lax
from jax.experimental import pallas as pl
from jax.experimental.pallas import tpu as pltpu
from jax.experimental.pallas import tpu_sc as plsc

_NODE_NUM = [1024, 768, 512, 1024, 896, 640, 1024, 1024]
_B = len(_NODE_NUM)
_D = 128
_MAXN = 1024
_MAXE = 3072
_MAXLEN = _MAXN + _MAXE + 1
_TOK = 4104

_NSEC = _B * _MAXN
_ESEC = _B * _MAXE
_GTOT = _NSEC + 2 * _ESEC

_NW = 32
_PW = _GTOT // _NW
_CH = 64
_NCH = _PW // _CH
_NB = 10
_RING = _NB + 1

_TOTAL_NODES = sum(_NODE_NUM)


def _sc_gather(k_idx, subnodes, orf0):
    mesh = plsc.VectorSubcoreMesh(core_axis_name="c", subcore_axis_name="s")

    @functools.partial(
        pl.kernel,
        out_type=jax.ShapeDtypeStruct((_GTOT, _D), jnp.float32),
        mesh=mesh,
        scratch_types=[
            pltpu.VMEM((_TOTAL_NODES,), jnp.int32),
            pltpu.VMEM((_PW,), jnp.int32),
            pltpu.VMEM((_PW,), jnp.int32),
            pltpu.VMEM((_RING, _CH, _D), jnp.float32),
            pltpu.SemaphoreType.DMA,
            pltpu.SemaphoreType.DMA,
        ],
        compiler_params=pltpu.CompilerParams(needs_layout_passes=False),
    )
    def run(k_hbm, sub_hbm, orf_hbm, out_hbm, sub_v, kv, fv, rows,
            gsem, dsem):
        wid = lax.axis_index("s") * 2 + lax.axis_index("c")
        base = wid * _PW
        pltpu.sync_copy(sub_hbm, sub_v)
        pltpu.sync_copy(k_hbm.at[pl.ds(base, _PW)], kv)

        def compose(i, carry):
            idx = kv[pl.ds(i * 16, 16)]
            fv[pl.ds(i * 16, 16)] = plsc.load_gather(sub_v, [idx])
            return carry

        lax.fori_loop(0, _PW // 16, compose, 0)

        def gath(c):
            return pltpu.async_copy(
                orf_hbm.at[fv.at[pl.ds(c * _CH, _CH)]],
                rows.at[c % _RING], gsem)

        def drain(c):
            return pltpu.async_copy(
                rows.at[c % _RING],
                out_hbm.at[pl.ds(base + c * _CH, _CH)], dsem)

        gcopies = [None] * _NCH
        dcopies = [None] * _NCH
        for c in range(_NB):
            gcopies[c] = gath(c)
        for c in range(_NCH):
            gcopies[c].wait()
            dcopies[c] = drain(c)
            nxt = c + _NB
            if nxt < _NCH:
                prev = nxt - _RING
                if prev >= 0:
                    dcopies[prev].wait()
                gcopies[nxt] = gath(nxt)
        for c in range(_NCH - _RING, _NCH):
            dcopies[c].wait()

    return run(k_idx, subnodes, orf0)


def _tc_body(num_ref, node_ref, edge_ref, gn_ref, gs_ref, gd_ref,
             es_ref, ed_ref, a1_ref, a2_ref, a12_ref, ow_ref,
             outf_ref, ps_ref, pd_ref):
    g = pl.program_id(0)
    n = num_ref[g]
    n3 = n * 3
    ow0 = ow_ref[0:1, :]
    ow1 = ow_ref[1:2, :]

    cn = (node_ref[...]
          + jnp.dot(gn_ref[...], a12_ref[...],
                    preferred_element_type=jnp.float32,
                    precision=lax.Precision.HIGHEST)
          + ow1)

    es = es_ref[...]
    ed = ed_ref[...]
    eqf = (es == ed).astype(jnp.float32)
    owe = ow0 + eqf * (ow1 - ow0)
    ce = (edge_ref[...]
          + jnp.dot(gs_ref[...], a1_ref[...],
                    preferred_element_type=jnp.float32,
                    precision=lax.Precision.HIGHEST)
          + jnp.dot(gd_ref[...], a2_ref[...],
                    preferred_element_type=jnp.float32,
                    precision=lax.Precision.HIGHEST)
          + owe)
    j = lax.broadcasted_iota(jnp.int32, (_MAXE, 1), 0)
    emask = j < n3
    ce = jnp.where(emask, ce, 0.0)

    outf_ref[...] = jnp.zeros((_TOK, _D), jnp.float32)
    outf_ref[0:_MAXN, :] = cn
    outf_ref[pl.ds(n, _MAXE), :] = ce

    t = lax.broadcasted_iota(jnp.int32, (_TOK, 1), 0)
    base = jnp.where(t < n, t, 0)
    zi = jnp.zeros((), jnp.int32)
    ps_ref[...] = base
    pd_ref[...] = base
    ps_ref[pl.ds(n, _MAXE), :] = jnp.where(emask, es, zi)
    pd_ref[pl.ds(n, _MAXE), :] = jnp.where(emask, ed, zi)


def _tc_call(num, node_p, edge_p, gn, gs, gd, es_col, ed_col, a1, a2, a12, ow):
    def per_graph(*blk):
        return pl.BlockSpec(blk, lambda g: (g,) + (0,) * (len(blk) - 1))

    def shared(*blk):
        return pl.BlockSpec(blk, lambda g: (0,) * len(blk))

    return pl.pallas_call(
        _tc_body,
        grid=(_B,),
        in_specs=[
            pl.BlockSpec(memory_space=pltpu.SMEM),
            per_graph(None, _MAXN, _D),
            per_graph(None, _MAXE, _D),
            per_graph(None, _MAXN, _D),
            per_graph(None, _MAXE, _D),
            per_graph(None, _MAXE, _D),
            per_graph(None, _MAXE, 1),
            per_graph(None, _MAXE, 1),
            shared(_D, _D),
            shared(_D, _D),
            shared(_D, _D),
            shared(2, _D),
        ],
        out_specs=[
            per_graph(None, _TOK, _D),
            per_graph(None, _TOK, 1),
            per_graph(None, _TOK, 1),
        ],
        out_shape=[
            jax.ShapeDtypeStruct((_B, _TOK, _D), jnp.float32),
            jax.ShapeDtypeStruct((_B, _TOK, 1), jnp.int32),
            jax.ShapeDtypeStruct((_B, _TOK, 1), jnp.int32),
        ],
        compiler_params=pltpu.CompilerParams(
            dimension_semantics=("arbitrary",)),
    )(num, node_p, edge_p, gn, gs, gd, es_col, ed_col, a1, a2, a12, ow)


def _static_setup():
    node_num = np.asarray(_NODE_NUM, np.int32)
    noff = np.concatenate([[0], np.cumsum(node_num)]).astype(np.int64)
    eoff = 3 * noff
    k1 = np.zeros((_B, _MAXN), np.int32)
    for g, n in enumerate(_NODE_NUM):
        k1[g, :n] = noff[g] + np.arange(n)
    tok = np.arange(_MAXLEN, dtype=np.int32)[None, :]
    nn = node_num[:, None]
    seq = 4 * nn
    padded_node_mask = tok < nn
    padded_edge_mask = (tok >= nn) & (tok < seq)
    padding_mask = tok >= seq
    return noff, eoff, k1.reshape(-1), padding_mask, padded_node_mask, padded_edge_mask


_NOFF, _EOFF, _K1, _PAD_MASK, _NODE_MASK, _EDGE_MASK = _static_setup()


def kernel(node_data, edge_data, edge_index, indices_subnodes,
           order_weight, orf_w, orf):
    orf0 = orf[0]
    k2, k3, es_cols, ed_cols, node_ps, edge_ps = [], [], [], [], [], []
    for g, n in enumerate(_NODE_NUM):
        e = 3 * n
        src = edge_index[0, _EOFF[g]:_EOFF[g] + e]
        dst = edge_index[1, _EOFF[g]:_EOFF[g] + e]
        k2.append(jnp.pad(src + np.int32(_NOFF[g]), (0, _MAXE - e)))
        k3.append(jnp.pad(dst + np.int32(_NOFF[g]), (0, _MAXE - e)))
        es_cols.append(jnp.pad(src, (0, _MAXE - e)))
        ed_cols.append(jnp.pad(dst, (0, _MAXE - e)))
        node_ps.append(jnp.pad(node_data[_NOFF[g]:_NOFF[g] + n],
                               ((0, _MAXN - n), (0, 0))))
        edge_ps.append(jnp.pad(edge_data[_EOFF[g]:_EOFF[g] + e],
                               ((0, _MAXE - e), (0, 0))))
    k_idx = jnp.concatenate([jnp.asarray(_K1)] + k2 + k3).astype(jnp.int32)
    node_p = jnp.stack(node_ps)
    edge_p = jnp.stack(edge_ps)
    es_col = jnp.stack(es_cols).astype(jnp.int32)[..., None]
    ed_col = jnp.stack(ed_cols).astype(jnp.int32)[..., None]

    gall = _sc_gather(k_idx, indices_subnodes, orf0)
    gn = gall[:_NSEC].reshape(_B, _MAXN, _D)
    gs = gall[_NSEC:_NSEC + _ESEC].reshape(_B, _MAXE, _D)
    gd = gall[_NSEC + _ESEC:].reshape(_B, _MAXE, _D)

    a1 = orf_w[:, :_D].T
    a2 = orf_w[:, _D:].T
    a12 = a1 + a2
    num = jnp.asarray(np.asarray(_NODE_NUM, np.int32))

    outf, ps, pd = _tc_call(num, node_p, edge_p, gn, gs, gd,
                            es_col, ed_col, a1, a2, a12, order_weight)

    padded_feature = outf[:, :_MAXLEN, :]
    padded_index = jnp.stack([ps[:, :_MAXLEN, 0], pd[:, :_MAXLEN, 0]], axis=-1)
    return (padded_feature,
            jnp.asarray(_PAD_MASK),
            padded_index,
            jnp.asarray(_NODE_MASK),
            jnp.asarray(_EDGE_MASK))


# speedup vs baseline: 3.7742x; 1.6853x over previous
import functools

import numpy as np

import jax
import jax.numpy as jnp
from jax import lax
from jax.experimental import pallas as pl
from jax.experimental.pallas import tpu as pltpu
from jax.experimental.pallas import tpu_sc as plsc

_NODE_NUM = [1024, 768, 512, 1024, 896, 640, 1024, 1024]
_B = len(_NODE_NUM)
_D = 128
_MAXN = 1024
_MAXE = 3072
_MAXLEN = _MAXN + _MAXE + 1
_TOK = 4104

_NSEC = _B * _MAXN

_NW = 32
_PW = _NSEC // _NW
_CH = 64
_NCH = _PW // _CH
_RTILE = 512

_TOTAL_NODES = sum(_NODE_NUM)


def _sc_gather(k_idx, subnodes, orf0):
    mesh = plsc.VectorSubcoreMesh(core_axis_name="c", subcore_axis_name="s")

    @functools.partial(
        pl.kernel,
        out_type=jax.ShapeDtypeStruct((_NSEC, _D), jnp.float32),
        mesh=mesh,
        scratch_types=[
            pltpu.VMEM((_TOTAL_NODES,), jnp.int32),
            pltpu.VMEM((_PW,), jnp.int32),
            pltpu.VMEM((_PW,), jnp.int32),
            pltpu.VMEM((_NCH, _CH, _D), jnp.float32),
            pltpu.SemaphoreType.DMA,
        ],
        compiler_params=pltpu.CompilerParams(needs_layout_passes=False),
    )
    def run(k_hbm, sub_hbm, orf_hbm, out_hbm, sub_v, kv, fv, rows, gsem):
        wid = lax.axis_index("s") * 2 + lax.axis_index("c")
        base = wid * _PW
        pltpu.sync_copy(sub_hbm, sub_v)
        pltpu.sync_copy(k_hbm.at[pl.ds(base, _PW)], kv)

        def compose(i, carry):
            idx = kv[pl.ds(i * 16, 16)]
            fv[pl.ds(i * 16, 16)] = plsc.load_gather(sub_v, [idx])
            return carry

        lax.fori_loop(0, _PW // 16, compose, 0)

        gcopies = []
        for c in range(_NCH):
            gcopies.append(pltpu.async_copy(
                orf_hbm.at[fv.at[pl.ds(c * _CH, _CH)]], rows.at[c], gsem))
        for c in range(_NCH):
            gcopies[c].wait()
            pltpu.sync_copy(rows.at[c],
                            out_hbm.at[pl.ds(base + c * _CH, _CH)])

    return run(k_idx, subnodes, orf0)


def _tc_body(num_ref, node_ref, edge_ref, gn_ref,
             es_ref, ed_ref, a1_ref, a2_ref, a12_ref, ow_ref,
             outf_ref, ps_ref, pd_ref):
    g = pl.program_id(0)
    n = num_ref[g]
    n3 = n * 3
    ow0 = ow_ref[0:1, :]
    ow1 = ow_ref[1:2, :]

    gn = gn_ref[...]
    p1 = jnp.dot(gn, a1_ref[...], preferred_element_type=jnp.float32,
                 precision=lax.Precision.HIGHEST)
    p2 = jnp.dot(gn, a2_ref[...], preferred_element_type=jnp.float32,
                 precision=lax.Precision.HIGHEST)
    pn = jnp.dot(gn, a12_ref[...], preferred_element_type=jnp.float32,
                 precision=lax.Precision.HIGHEST)

    outf_ref[...] = jnp.zeros((_TOK, _D), jnp.float32)
    outf_ref[0:_MAXN, :] = node_ref[...] + pn + ow1

    tab = lax.broadcasted_iota(jnp.int32, (1, _MAXN), 1)
    for r in range(_MAXE // _RTILE):
        lo = r * _RTILE
        es_r = es_ref[lo:lo + _RTILE, :]
        ed_r = ed_ref[lo:lo + _RTILE, :]
        ohs = (es_r == tab).astype(jnp.float32)
        ohd = (ed_r == tab).astype(jnp.float32)
        eqf = (es_r == ed_r).astype(jnp.float32)
        owe = ow0 + eqf * (ow1 - ow0)
        ce = (edge_ref[lo:lo + _RTILE, :]
              + jnp.dot(ohs, p1, preferred_element_type=jnp.float32,
                        precision=lax.Precision.HIGHEST)
              + jnp.dot(ohd, p2, preferred_element_type=jnp.float32,
                        precision=lax.Precision.HIGHEST)
              + owe)
        jr = lax.broadcasted_iota(jnp.int32, (_RTILE, 1), 0) + lo
        ce = jnp.where(jr < n3, ce, 0.0)
        outf_ref[pl.ds(n + lo, _RTILE), :] = ce

    j = lax.broadcasted_iota(jnp.int32, (_MAXE, 1), 0)
    emask = j < n3
    t = lax.broadcasted_iota(jnp.int32, (_TOK, 1), 0)
    base = jnp.where(t < n, t, 0)
    zi = jnp.zeros((), jnp.int32)
    ps_ref[...] = base
    pd_ref[...] = base
    ps_ref[pl.ds(n, _MAXE), :] = jnp.where(emask, es_ref[...], zi)
    pd_ref[pl.ds(n, _MAXE), :] = jnp.where(emask, ed_ref[...], zi)


def _tc_call(num, node_p, edge_p, gn, es_col, ed_col, a1, a2, a12, ow):
    def per_graph(*blk):
        return pl.BlockSpec(blk, lambda g: (g,) + (0,) * (len(blk) - 1))

    def shared(*blk):
        return pl.BlockSpec(blk, lambda g: (0,) * len(blk))

    return pl.pallas_call(
        _tc_body,
        grid=(_B,),
        in_specs=[
            pl.BlockSpec(memory_space=pltpu.SMEM),
            per_graph(None, _MAXN, _D),
            per_graph(None, _MAXE, _D),
            per_graph(None, _MAXN, _D),
            per_graph(None, _MAXE, 1),
            per_graph(None, _MAXE, 1),
            shared(_D, _D),
            shared(_D, _D),
            shared(_D, _D),
            shared(2, _D),
        ],
        out_specs=[
            per_graph(None, _TOK, _D),
            per_graph(None, _TOK, 1),
            per_graph(None, _TOK, 1),
        ],
        out_shape=[
            jax.ShapeDtypeStruct((_B, _TOK, _D), jnp.float32),
            jax.ShapeDtypeStruct((_B, _TOK, 1), jnp.int32),
            jax.ShapeDtypeStruct((_B, _TOK, 1), jnp.int32),
        ],
        compiler_params=pltpu.CompilerParams(
            dimension_semantics=("arbitrary",)),
    )(num, node_p, edge_p, gn, es_col, ed_col, a1, a2, a12, ow)


def _static_setup():
    node_num = np.asarray(_NODE_NUM, np.int32)
    noff = np.concatenate([[0], np.cumsum(node_num)]).astype(np.int64)
    eoff = 3 * noff
    k1 = np.zeros((_B, _MAXN), np.int32)
    for g, n in enumerate(_NODE_NUM):
        k1[g, :n] = noff[g] + np.arange(n)
    tok = np.arange(_MAXLEN, dtype=np.int32)[None, :]
    nn = node_num[:, None]
    seq = 4 * nn
    padded_node_mask = tok < nn
    padded_edge_mask = (tok >= nn) & (tok < seq)
    padding_mask = tok >= seq
    return noff, eoff, k1.reshape(-1), padding_mask, padded_node_mask, padded_edge_mask


_NOFF, _EOFF, _K1, _PAD_MASK, _NODE_MASK, _EDGE_MASK = _static_setup()


def kernel(node_data, edge_data, edge_index, indices_subnodes,
           order_weight, orf_w, orf):
    orf0 = orf[0]
    es_cols, ed_cols, node_ps, edge_ps = [], [], [], []
    for g, n in enumerate(_NODE_NUM):
        e = 3 * n
        src = edge_index[0, _EOFF[g]:_EOFF[g] + e]
        dst = edge_index[1, _EOFF[g]:_EOFF[g] + e]
        es_cols.append(jnp.pad(src, (0, _MAXE - e)))
        ed_cols.append(jnp.pad(dst, (0, _MAXE - e)))
        node_ps.append(jnp.pad(node_data[_NOFF[g]:_NOFF[g] + n],
                               ((0, _MAXN - n), (0, 0))))
        edge_ps.append(jnp.pad(edge_data[_EOFF[g]:_EOFF[g] + e],
                               ((0, _MAXE - e), (0, 0))))
    k_idx = jnp.asarray(_K1)
    node_p = jnp.stack(node_ps)
    edge_p = jnp.stack(edge_ps)
    es_col = jnp.stack(es_cols).astype(jnp.int32)[..., None]
    ed_col = jnp.stack(ed_cols).astype(jnp.int32)[..., None]

    gn = _sc_gather(k_idx, indices_subnodes, orf0).reshape(_B, _MAXN, _D)

    a1 = orf_w[:, :_D].T
    a2 = orf_w[:, _D:].T
    a12 = a1 + a2
    num = jnp.asarray(np.asarray(_NODE_NUM, np.int32))

    outf, ps, pd = _tc_call(num, node_p, edge_p, gn,
                            es_col, ed_col, a1, a2, a12, order_weight)

    padded_feature = outf[:, :_MAXLEN, :]
    padded_index = jnp.stack([ps[:, :_MAXLEN, 0], pd[:, :_MAXLEN, 0]], axis=-1)
    return (padded_feature,
            jnp.asarray(_PAD_MASK),
            padded_index,
            jnp.asarray(_NODE_MASK),
            jnp.asarray(_EDGE_MASK))


# speedup vs baseline: 5.1142x; 1.3551x over previous
import functools

import numpy as np

import jax
import jax.numpy as jnp
from jax import lax
from jax.experimental import pallas as pl
from jax.experimental.pallas import tpu as pltpu
from jax.experimental.pallas import tpu_sc as plsc

_NODE_NUM = [1024, 768, 512, 1024, 896, 640, 1024, 1024]
_B = len(_NODE_NUM)
_D = 128
_MAXN = 1024
_MAXE = 3072
_MAXLEN = _MAXN + _MAXE + 1

_NSEC = _B * _MAXN

_NW = 32
_PW = _NSEC // _NW
_CH = 64
_NCH = _PW // _CH
_RTILE = 512

_TOTAL_NODES = sum(_NODE_NUM)


def _sc_gather(slot_idx, orf0):
    mesh = plsc.VectorSubcoreMesh(core_axis_name="c", subcore_axis_name="s")

    @functools.partial(
        pl.kernel,
        out_type=jax.ShapeDtypeStruct((_NSEC, _D), jnp.float32),
        mesh=mesh,
        scratch_types=[
            pltpu.VMEM((_PW,), jnp.int32),
            pltpu.VMEM((_NCH, _CH, _D), jnp.float32),
            pltpu.SemaphoreType.DMA,
        ],
        compiler_params=pltpu.CompilerParams(needs_layout_passes=False),
    )
    def run(idx_hbm, orf_hbm, out_hbm, kv, rows, gsem):
        wid = lax.axis_index("s") * 2 + lax.axis_index("c")
        base = wid * _PW
        pltpu.sync_copy(idx_hbm.at[pl.ds(base, _PW)], kv)
        gcopies = []
        for c in range(_NCH):
            gcopies.append(pltpu.async_copy(
                orf_hbm.at[kv.at[pl.ds(c * _CH, _CH)]], rows.at[c], gsem))
        for c in range(_NCH):
            gcopies[c].wait()
            pltpu.sync_copy(rows.at[c],
                            out_hbm.at[pl.ds(base + c * _CH, _CH)])

    return run(slot_idx, orf0)


def _split_bf16(x):
    hi = x.astype(jnp.bfloat16)
    lo = (x - hi.astype(jnp.float32)).astype(jnp.bfloat16)
    return hi, lo


def _tc_body(num_ref, node_ref, edge_ref, gn_ref,
             es_ref, ed_ref, a1_ref, a2_ref, a12_ref, ow_ref,
             outf_ref, ps_ref, pd_ref):
    g = pl.program_id(0)
    n = num_ref[g]
    n3 = n * 3
    ow0 = ow_ref[0:1, :]
    ow1 = ow_ref[1:2, :]

    gn = gn_ref[...]
    p1 = jnp.dot(gn, a1_ref[...], preferred_element_type=jnp.float32,
                 precision=lax.Precision.HIGHEST)
    p2 = jnp.dot(gn, a2_ref[...], preferred_element_type=jnp.float32,
                 precision=lax.Precision.HIGHEST)
    pn = jnp.dot(gn, a12_ref[...], preferred_element_type=jnp.float32,
                 precision=lax.Precision.HIGHEST)
    p1h, p1l = _split_bf16(p1)
    p2h, p2l = _split_bf16(p2)

    outf_ref[...] = jnp.zeros((_MAXLEN, _D), jnp.float32)
    outf_ref[0:_MAXN, :] = node_ref[...] + pn + ow1

    tab = lax.broadcasted_iota(jnp.int32, (1, _MAXN), 1)
    for r in range(_MAXE // _RTILE):
        lo = r * _RTILE
        es_r = es_ref[lo:lo + _RTILE, :]
        ed_r = ed_ref[lo:lo + _RTILE, :]
        ohs = (es_r == tab).astype(jnp.float32).astype(jnp.bfloat16)
        ohd = (ed_r == tab).astype(jnp.float32).astype(jnp.bfloat16)
        eqf = (es_r == ed_r).astype(jnp.float32)
        owe = ow0 + eqf * (ow1 - ow0)
        acc = (jnp.dot(ohs, p1h, preferred_element_type=jnp.float32)
               + jnp.dot(ohs, p1l, preferred_element_type=jnp.float32)
               + jnp.dot(ohd, p2h, preferred_element_type=jnp.float32)
               + jnp.dot(ohd, p2l, preferred_element_type=jnp.float32))
        ce = edge_ref[lo:lo + _RTILE, :] + acc + owe
        jr = lax.broadcasted_iota(jnp.int32, (_RTILE, 1), 0) + lo
        ce = jnp.where(jr < n3, ce, 0.0)
        outf_ref[pl.ds(n + lo, _RTILE), :] = ce

    j = lax.broadcasted_iota(jnp.int32, (_MAXE, 1), 0)
    emask = j < n3
    t = lax.broadcasted_iota(jnp.int32, (_MAXLEN, 1), 0)
    base = jnp.where(t < n, t, 0)
    zi = jnp.zeros((), jnp.int32)
    ps_ref[...] = base
    pd_ref[...] = base
    ps_ref[pl.ds(n, _MAXE), :] = jnp.where(emask, es_ref[...], zi)
    pd_ref[pl.ds(n, _MAXE), :] = jnp.where(emask, ed_ref[...], zi)


def _tc_call(num, node_p, edge_p, gn, es_col, ed_col, a1, a2, a12, ow):
    def per_graph(*blk):
        return pl.BlockSpec(blk, lambda g: (g,) + (0,) * (len(blk) - 1))

    def shared(*blk):
        return pl.BlockSpec(blk, lambda g: (0,) * len(blk))

    return pl.pallas_call(
        _tc_body,
        grid=(_B,),
        in_specs=[
            pl.BlockSpec(memory_space=pltpu.SMEM),
            per_graph(None, _MAXN, _D),
            per_graph(None, _MAXE, _D),
            per_graph(None, _MAXN, _D),
            per_graph(None, _MAXE, 1),
            per_graph(None, _MAXE, 1),
            shared(_D, _D),
            shared(_D, _D),
            shared(_D, _D),
            shared(2, _D),
        ],
        out_specs=[
            per_graph(None, _MAXLEN, _D),
            per_graph(None, _MAXLEN, 1),
            per_graph(None, _MAXLEN, 1),
        ],
        out_shape=[
            jax.ShapeDtypeStruct((_B, _MAXLEN, _D), jnp.float32),
            jax.ShapeDtypeStruct((_B, _MAXLEN, 1), jnp.int32),
            jax.ShapeDtypeStruct((_B, _MAXLEN, 1), jnp.int32),
        ],
        compiler_params=pltpu.CompilerParams(
            dimension_semantics=("arbitrary",)),
    )(num, node_p, edge_p, gn, es_col, ed_col, a1, a2, a12, ow)


def _static_setup():
    node_num = np.asarray(_NODE_NUM, np.int32)
    noff = np.concatenate([[0], np.cumsum(node_num)]).astype(np.int64)
    eoff = 3 * noff
    tok = np.arange(_MAXLEN, dtype=np.int32)[None, :]
    nn = node_num[:, None]
    seq = 4 * nn
    padded_node_mask = tok < nn
    padded_edge_mask = (tok >= nn) & (tok < seq)
    padding_mask = tok >= seq
    return noff, eoff, padding_mask, padded_node_mask, padded_edge_mask


_NOFF, _EOFF, _PAD_MASK, _NODE_MASK, _EDGE_MASK = _static_setup()


def kernel(node_data, edge_data, edge_index, indices_subnodes,
           order_weight, orf_w, orf):
    orf0 = orf[0]
    es_cols, ed_cols, node_ps, edge_ps, sub_ps = [], [], [], [], []
    for g, n in enumerate(_NODE_NUM):
        e = 3 * n
        src = edge_index[0, _EOFF[g]:_EOFF[g] + e]
        dst = edge_index[1, _EOFF[g]:_EOFF[g] + e]
        es_cols.append(jnp.pad(src, (0, _MAXE - e)))
        ed_cols.append(jnp.pad(dst, (0, _MAXE - e)))
        node_ps.append(jnp.pad(node_data[_NOFF[g]:_NOFF[g] + n],
                               ((0, _MAXN - n), (0, 0))))
        edge_ps.append(jnp.pad(edge_data[_EOFF[g]:_EOFF[g] + e],
                               ((0, _MAXE - e), (0, 0))))
        sub_ps.append(jnp.pad(indices_subnodes[_NOFF[g]:_NOFF[g] + n],
                              (0, _MAXN - n)))
    slot_idx = jnp.concatenate(sub_ps).astype(jnp.int32)
    node_p = jnp.stack(node_ps)
    edge_p = jnp.stack(edge_ps)
    es_col = jnp.stack(es_cols).astype(jnp.int32)[..., None]
    ed_col = jnp.stack(ed_cols).astype(jnp.int32)[..., None]

    gn = _sc_gather(slot_idx, orf0).reshape(_B, _MAXN, _D)

    a1 = orf_w[:, :_D].T
    a2 = orf_w[:, _D:].T
    a12 = a1 + a2
    num = jnp.asarray(np.asarray(_NODE_NUM, np.int32))

    padded_feature, ps, pd = _tc_call(num, node_p, edge_p, gn,
                                      es_col, ed_col, a1, a2, a12,
                                      order_weight)

    padded_index = jnp.stack([ps[..., 0], pd[..., 0]], axis=-1)
    return (padded_feature,
            jnp.asarray(_PAD_MASK),
            padded_index,
            jnp.asarray(_NODE_MASK),
            jnp.asarray(_EDGE_MASK))


# speedup vs baseline: 5.8395x; 1.1418x over previous
import functools

import numpy as np

import jax
import jax.numpy as jnp
from jax import lax
from jax.experimental import pallas as pl
from jax.experimental.pallas import tpu as pltpu
from jax.experimental.pallas import tpu_sc as plsc

_NODE_NUM = [1024, 768, 512, 1024, 896, 640, 1024, 1024]
_B = len(_NODE_NUM)
_D = 128
_MAXN = 1024
_MAXE = 3072
_MAXLEN = _MAXN + _MAXE + 1

_NSEC = _B * _MAXN

_NW = 32
_PW = _NSEC // _NW
_CH = 64
_NCH = _PW // _CH
_RTILE = 512

_TOTAL_NODES = sum(_NODE_NUM)


def _sc_gather(slot_idx, orf0):
    mesh = plsc.VectorSubcoreMesh(core_axis_name="c", subcore_axis_name="s")

    @functools.partial(
        pl.kernel,
        out_type=jax.ShapeDtypeStruct((_NSEC, _D), jnp.float32),
        mesh=mesh,
        scratch_types=[
            pltpu.VMEM((_PW,), jnp.int32),
            pltpu.VMEM((_NCH, _CH, _D), jnp.float32),
            pltpu.SemaphoreType.DMA,
        ],
        compiler_params=pltpu.CompilerParams(needs_layout_passes=False),
    )
    def run(idx_hbm, orf_hbm, out_hbm, kv, rows, gsem):
        wid = lax.axis_index("s") * 2 + lax.axis_index("c")
        base = wid * _PW
        pltpu.sync_copy(idx_hbm.at[pl.ds(base, _PW)], kv)
        gcopies = []
        for c in range(_NCH):
            gcopies.append(pltpu.async_copy(
                orf_hbm.at[kv.at[pl.ds(c * _CH, _CH)]], rows.at[c], gsem))
        for c in range(_NCH):
            gcopies[c].wait()
            pltpu.sync_copy(rows.at[c],
                            out_hbm.at[pl.ds(base + c * _CH, _CH)])

    return run(slot_idx, orf0)


def _split_bf16(x):
    hi = x.astype(jnp.bfloat16)
    lo = (x - hi.astype(jnp.float32)).astype(jnp.bfloat16)
    return hi, lo


def _tc_body(num_ref, node_ref, edge_ref, gn_ref,
             es_ref, ed_ref, a1_ref, a2_ref, a12_ref, ow_ref,
             outf_ref, ps_ref, pd_ref):
    g = pl.program_id(0)
    n = num_ref[g]
    n3 = n * 3
    ow0 = ow_ref[0:1, :]
    ow1 = ow_ref[1:2, :]

    gn = gn_ref[...]
    p1 = jnp.dot(gn, a1_ref[...], preferred_element_type=jnp.float32,
                 precision=lax.Precision.HIGHEST)
    p2 = jnp.dot(gn, a2_ref[...], preferred_element_type=jnp.float32,
                 precision=lax.Precision.HIGHEST)
    pn = jnp.dot(gn, a12_ref[...], preferred_element_type=jnp.float32,
                 precision=lax.Precision.HIGHEST)
    p1h, p1l = _split_bf16(p1)
    p2h, p2l = _split_bf16(p2)
    rhs = jnp.concatenate(
        [jnp.concatenate([p1h, p1l], axis=1),
         jnp.concatenate([p2h, p2l], axis=1)], axis=0)

    outf_ref[...] = jnp.zeros((_MAXLEN, _D), jnp.float32)
    outf_ref[0:_MAXN, :] = node_ref[...] + pn + ow1

    tab16 = lax.broadcasted_iota(jnp.int16, (1, _MAXN), 1)
    oneb = jnp.ones((), jnp.bfloat16)
    zerob = jnp.zeros((), jnp.bfloat16)
    for r in range(_MAXE // _RTILE):
        lo = r * _RTILE
        es_r = es_ref[lo:lo + _RTILE, :]
        ed_r = ed_ref[lo:lo + _RTILE, :]
        ohs = jnp.where(es_r.astype(jnp.int16) == tab16, oneb, zerob)
        ohd = jnp.where(ed_r.astype(jnp.int16) == tab16, oneb, zerob)
        oh = jnp.concatenate([ohs, ohd], axis=1)
        eqf = (es_r == ed_r).astype(jnp.float32)
        owe = ow0 + eqf * (ow1 - ow0)
        res = jnp.dot(oh, rhs, preferred_element_type=jnp.float32)
        acc = res[:, :_D] + res[:, _D:]
        ce = edge_ref[lo:lo + _RTILE, :] + acc + owe
        jr = lax.broadcasted_iota(jnp.int32, (_RTILE, 1), 0) + lo
        ce = jnp.where(jr < n3, ce, 0.0)
        outf_ref[pl.ds(n + lo, _RTILE), :] = ce

    j = lax.broadcasted_iota(jnp.int32, (_MAXE, 1), 0)
    emask = j < n3
    t = lax.broadcasted_iota(jnp.int32, (_MAXLEN, 1), 0)
    base = jnp.where(t < n, t, 0)
    zi = jnp.zeros((), jnp.int32)
    ps_ref[...] = base
    pd_ref[...] = base
    ps_ref[pl.ds(n, _MAXE), :] = jnp.where(emask, es_ref[...], zi)
    pd_ref[pl.ds(n, _MAXE), :] = jnp.where(emask, ed_ref[...], zi)


def _tc_call(num, node_p, edge_p, gn, es_col, ed_col, a1, a2, a12, ow):
    def per_graph(*blk):
        return pl.BlockSpec(blk, lambda g: (g,) + (0,) * (len(blk) - 1))

    def shared(*blk):
        return pl.BlockSpec(blk, lambda g: (0,) * len(blk))

    return pl.pallas_call(
        _tc_body,
        grid=(_B,),
        in_specs=[
            pl.BlockSpec(memory_space=pltpu.SMEM),
            per_graph(None, _MAXN, _D),
            per_graph(None, _MAXE, _D),
            per_graph(None, _MAXN, _D),
            per_graph(None, _MAXE, 1),
            per_graph(None, _MAXE, 1),
            shared(_D, _D),
            shared(_D, _D),
            shared(_D, _D),
            shared(2, _D),
        ],
        out_specs=[
            per_graph(None, _MAXLEN, _D),
            per_graph(None, _MAXLEN, 1),
            per_graph(None, _MAXLEN, 1),
        ],
        out_shape=[
            jax.ShapeDtypeStruct((_B, _MAXLEN, _D), jnp.float32),
            jax.ShapeDtypeStruct((_B, _MAXLEN, 1), jnp.int32),
            jax.ShapeDtypeStruct((_B, _MAXLEN, 1), jnp.int32),
        ],
        compiler_params=pltpu.CompilerParams(
            dimension_semantics=("arbitrary",)),
    )(num, node_p, edge_p, gn, es_col, ed_col, a1, a2, a12, ow)


def _static_setup():
    node_num = np.asarray(_NODE_NUM, np.int32)
    noff = np.concatenate([[0], np.cumsum(node_num)]).astype(np.int64)
    eoff = 3 * noff
    tok = np.arange(_MAXLEN, dtype=np.int32)[None, :]
    nn = node_num[:, None]
    seq = 4 * nn
    padded_node_mask = tok < nn
    padded_edge_mask = (tok >= nn) & (tok < seq)
    padding_mask = tok >= seq
    return noff, eoff, padding_mask, padded_node_mask, padded_edge_mask


_NOFF, _EOFF, _PAD_MASK, _NODE_MASK, _EDGE_MASK = _static_setup()


def kernel(node_data, edge_data, edge_index, indices_subnodes,
           order_weight, orf_w, orf):
    orf0 = orf[0]
    es_cols, ed_cols, node_ps, edge_ps, sub_ps = [], [], [], [], []
    for g, n in enumerate(_NODE_NUM):
        e = 3 * n
        src = edge_index[0, _EOFF[g]:_EOFF[g] + e]
        dst = edge_index[1, _EOFF[g]:_EOFF[g] + e]
        es_cols.append(jnp.pad(src, (0, _MAXE - e)))
        ed_cols.append(jnp.pad(dst, (0, _MAXE - e)))
        node_ps.append(jnp.pad(node_data[_NOFF[g]:_NOFF[g] + n],
                               ((0, _MAXN - n), (0, 0))))
        edge_ps.append(jnp.pad(edge_data[_EOFF[g]:_EOFF[g] + e],
                               ((0, _MAXE - e), (0, 0))))
        sub_ps.append(jnp.pad(indices_subnodes[_NOFF[g]:_NOFF[g] + n],
                              (0, _MAXN - n)))
    slot_idx = jnp.concatenate(sub_ps).astype(jnp.int32)
    node_p = jnp.stack(node_ps)
    edge_p = jnp.stack(edge_ps)
    es_col = jnp.stack(es_cols).astype(jnp.int32)[..., None]
    ed_col = jnp.stack(ed_cols).astype(jnp.int32)[..., None]

    gn = _sc_gather(slot_idx, orf0).reshape(_B, _MAXN, _D)

    a1 = orf_w[:, :_D].T
    a2 = orf_w[:, _D:].T
    a12 = a1 + a2
    num = jnp.asarray(np.asarray(_NODE_NUM, np.int32))

    padded_feature, ps, pd = _tc_call(num, node_p, edge_p, gn,
                                      es_col, ed_col, a1, a2, a12,
                                      order_weight)

    padded_index = jnp.stack([ps[..., 0], pd[..., 0]], axis=-1)
    return (padded_feature,
            jnp.asarray(_PAD_MASK),
            padded_index,
            jnp.asarray(_NODE_MASK),
            jnp.asarray(_EDGE_MASK))


# speedup vs baseline: 7.8709x; 1.3479x over previous
import functools

import numpy as np

import jax
import jax.numpy as jnp
from jax import lax
from jax.experimental import pallas as pl
from jax.experimental.pallas import tpu as pltpu
from jax.experimental.pallas import tpu_sc as plsc

_NODE_NUM = [1024, 768, 512, 1024, 896, 640, 1024, 1024]
_B = len(_NODE_NUM)
_D = 128
_MAXN = 1024
_MAXE = 3072
_MAXLEN = _MAXN + _MAXE + 1

_TOTAL_NODES = sum(_NODE_NUM)
_TOTAL_EDGES = 3 * _TOTAL_NODES

_NW = 32
_PW = _TOTAL_NODES // _NW
_CH = 72
_NCH = _PW // _CH
_RTILE = 512


def _sc_gather(slot_idx, orf0):
    mesh = plsc.VectorSubcoreMesh(core_axis_name="c", subcore_axis_name="s")

    @functools.partial(
        pl.kernel,
        out_type=jax.ShapeDtypeStruct((_TOTAL_NODES, _D), jnp.float32),
        mesh=mesh,
        scratch_types=[
            pltpu.VMEM((_PW,), jnp.int32),
            pltpu.VMEM((_NCH, _CH, _D), jnp.float32),
            pltpu.SemaphoreType.DMA,
        ],
        compiler_params=pltpu.CompilerParams(needs_layout_passes=False),
    )
    def run(idx_hbm, orf_hbm, out_hbm, kv, rows, gsem):
        wid = lax.axis_index("s") * 2 + lax.axis_index("c")
        base = wid * _PW
        pltpu.sync_copy(idx_hbm.at[pl.ds(base, _PW)], kv)
        gcopies = []
        for c in range(_NCH):
            gcopies.append(pltpu.async_copy(
                orf_hbm.at[kv.at[pl.ds(c * _CH, _CH)]], rows.at[c], gsem))
        for c in range(_NCH):
            gcopies[c].wait()
            pltpu.sync_copy(rows.at[c],
                            out_hbm.at[pl.ds(base + c * _CH, _CH)])

    return run(slot_idx, orf0)


def _split_bf16(x):
    hi = x.astype(jnp.bfloat16)
    lo = (x - hi.astype(jnp.float32)).astype(jnp.bfloat16)
    return hi, lo


def _tc_body(num_ref, noff_ref, eoff_ref, node_hbm, edge_hbm, gn_hbm,
             es_ref, ed_ref, a1_ref, a2_ref, a12_ref, ow_ref,
             outf_ref, ps_ref, pd_ref,
             node_v, edge_v, gn_v, nsem, esem, gsem):
    g = pl.program_id(0)
    n = num_ref[g]
    no = noff_ref[g]
    eo = eoff_ref[g]
    n3 = n * 3
    ow0 = ow_ref[0:1, :]
    ow1 = ow_ref[1:2, :]

    cp_g = pltpu.make_async_copy(gn_hbm.at[pl.ds(no, _MAXN)], gn_v, gsem)
    cp_n = pltpu.make_async_copy(node_hbm.at[pl.ds(no, _MAXN)], node_v, nsem)
    cp_e = pltpu.make_async_copy(edge_hbm.at[pl.ds(eo, _MAXE)], edge_v, esem)
    cp_g.start()
    cp_n.start()
    cp_e.start()

    j = lax.broadcasted_iota(jnp.int32, (_MAXE, 1), 0)
    emask = j < n3
    t = lax.broadcasted_iota(jnp.int32, (_MAXLEN, 1), 0)
    base = jnp.where(t < n, t, 0)
    zi = jnp.zeros((), jnp.int32)
    ps_ref[...] = base
    pd_ref[...] = base
    ps_ref[pl.ds(n, _MAXE), :] = jnp.where(emask, es_ref[...], zi)
    pd_ref[pl.ds(n, _MAXE), :] = jnp.where(emask, ed_ref[...], zi)

    cp_g.wait()
    gn = gn_v[...]
    p1 = jnp.dot(gn, a1_ref[...], preferred_element_type=jnp.float32,
                 precision=lax.Precision.HIGHEST)
    p2 = jnp.dot(gn, a2_ref[...], preferred_element_type=jnp.float32,
                 precision=lax.Precision.HIGHEST)
    pn = jnp.dot(gn, a12_ref[...], preferred_element_type=jnp.float32,
                 precision=lax.Precision.HIGHEST)
    p1h, p1l = _split_bf16(p1)
    p2h, p2l = _split_bf16(p2)
    rhs = jnp.concatenate(
        [jnp.concatenate([p1h, p1l], axis=1),
         jnp.concatenate([p2h, p2l], axis=1)], axis=0)

    cp_n.wait()
    outf_ref[...] = jnp.zeros((_MAXLEN, _D), jnp.float32)
    outf_ref[0:_MAXN, :] = node_v[...] + pn + ow1
    cp_e.wait()

    tab16 = lax.broadcasted_iota(jnp.int16, (1, _MAXN), 1)
    oneb = jnp.ones((), jnp.bfloat16)
    zerob = jnp.zeros((), jnp.bfloat16)
    for r in range(_MAXE // _RTILE):
        lo = r * _RTILE
        es_r = es_ref[lo:lo + _RTILE, :]
        ed_r = ed_ref[lo:lo + _RTILE, :]
        ohs = jnp.where(es_r.astype(jnp.int16) == tab16, oneb, zerob)
        ohd = jnp.where(ed_r.astype(jnp.int16) == tab16, oneb, zerob)
        oh = jnp.concatenate([ohs, ohd], axis=1)
        eqf = (es_r == ed_r).astype(jnp.float32)
        owe = ow0 + eqf * (ow1 - ow0)
        res = jnp.dot(oh, rhs, preferred_element_type=jnp.float32)
        acc = res[:, :_D] + res[:, _D:]
        ce = edge_v[lo:lo + _RTILE, :] + acc + owe
        jr = lax.broadcasted_iota(jnp.int32, (_RTILE, 1), 0) + lo
        ce = jnp.where(jr < n3, ce, 0.0)
        outf_ref[pl.ds(n + lo, _RTILE), :] = ce


def _tc_call(num, noff, eoff, node_data, edge_data, gn,
             es_col, ed_col, a1, a2, a12, ow):
    def per_graph(*blk):
        return pl.BlockSpec(blk, lambda g: (g,) + (0,) * (len(blk) - 1))

    def shared(*blk):
        return pl.BlockSpec(blk, lambda g: (0,) * len(blk))

    smem = pl.BlockSpec(memory_space=pltpu.SMEM)
    hbm = pl.BlockSpec(memory_space=pl.ANY)
    return pl.pallas_call(
        _tc_body,
        grid=(_B,),
        in_specs=[
            smem, smem, smem,
            hbm, hbm, hbm,
            per_graph(None, _MAXE, 1),
            per_graph(None, _MAXE, 1),
            shared(_D, _D),
            shared(_D, _D),
            shared(_D, _D),
            shared(2, _D),
        ],
        out_specs=[
            per_graph(None, _MAXLEN, _D),
            per_graph(None, _MAXLEN, 1),
            per_graph(None, _MAXLEN, 1),
        ],
        out_shape=[
            jax.ShapeDtypeStruct((_B, _MAXLEN, _D), jnp.float32),
            jax.ShapeDtypeStruct((_B, _MAXLEN, 1), jnp.int32),
            jax.ShapeDtypeStruct((_B, _MAXLEN, 1), jnp.int32),
        ],
        scratch_shapes=[
            pltpu.VMEM((_MAXN, _D), jnp.float32),
            pltpu.VMEM((_MAXE, _D), jnp.float32),
            pltpu.VMEM((_MAXN, _D), jnp.float32),
            pltpu.SemaphoreType.DMA,
            pltpu.SemaphoreType.DMA,
            pltpu.SemaphoreType.DMA,
        ],
        compiler_params=pltpu.CompilerParams(
            dimension_semantics=("arbitrary",)),
    )(num, noff, eoff, node_data, edge_data, gn,
      es_col, ed_col, a1, a2, a12, ow)


def _static_setup():
    node_num = np.asarray(_NODE_NUM, np.int32)
    noff = np.concatenate([[0], np.cumsum(node_num)]).astype(np.int64)
    eoff = 3 * noff
    tok = np.arange(_MAXLEN, dtype=np.int32)[None, :]
    nn = node_num[:, None]
    seq = 4 * nn
    padded_node_mask = tok < nn
    padded_edge_mask = (tok >= nn) & (tok < seq)
    padding_mask = tok >= seq
    return noff, eoff, padding_mask, padded_node_mask, padded_edge_mask


_NOFF, _EOFF, _PAD_MASK, _NODE_MASK, _EDGE_MASK = _static_setup()


def kernel(node_data, edge_data, edge_index, indices_subnodes,
           order_weight, orf_w, orf):
    orf0 = orf[0]
    es_cols, ed_cols = [], []
    for g, n in enumerate(_NODE_NUM):
        e = 3 * n
        es_cols.append(jnp.pad(edge_index[0, _EOFF[g]:_EOFF[g] + e],
                               (0, _MAXE - e)))
        ed_cols.append(jnp.pad(edge_index[1, _EOFF[g]:_EOFF[g] + e],
                               (0, _MAXE - e)))
    es_col = jnp.stack(es_cols).astype(jnp.int32)[..., None]
    ed_col = jnp.stack(ed_cols).astype(jnp.int32)[..., None]

    gn = _sc_gather(indices_subnodes.astype(jnp.int32), orf0)

    a1 = orf_w[:, :_D].T
    a2 = orf_w[:, _D:].T
    a12 = a1 + a2
    num = jnp.asarray(np.asarray(_NODE_NUM, np.int32))
    noff = jnp.asarray(_NOFF[:_B].astype(np.int32))
    eoff = jnp.asarray(_EOFF[:_B].astype(np.int32))

    padded_feature, ps, pd = _tc_call(num, noff, eoff,
                                      node_data, edge_data, gn,
                                      es_col, ed_col, a1, a2, a12,
                                      order_weight)

    padded_index = jnp.stack([ps[..., 0], pd[..., 0]], axis=-1)
    return (padded_feature,
            jnp.asarray(_PAD_MASK),
            padded_index,
            jnp.asarray(_NODE_MASK),
            jnp.asarray(_EDGE_MASK))


# speedup vs baseline: 9.0584x; 1.1509x over previous
import functools

import numpy as np

import jax
import jax.numpy as jnp
from jax import lax
from jax.experimental import pallas as pl
from jax.experimental.pallas import tpu as pltpu
from jax.experimental.pallas import tpu_sc as plsc

_NODE_NUM = [1024, 768, 512, 1024, 896, 640, 1024, 1024]
_B = len(_NODE_NUM)
_D = 128
_MAXN = 1024
_MAXE = 3072
_MAXLEN = _MAXN + _MAXE + 1

_TOTAL_NODES = sum(_NODE_NUM)
_TOTAL_EDGES = 3 * _TOTAL_NODES

_NW = 32
_PW = _TOTAL_NODES // _NW
_CH = 72
_NCH = _PW // _CH
_RTILE = 512


def _sc_gather(slot_idx, orf0):
    mesh = plsc.VectorSubcoreMesh(core_axis_name="c", subcore_axis_name="s")

    @functools.partial(
        pl.kernel,
        out_type=jax.ShapeDtypeStruct((_TOTAL_NODES, _D), jnp.float32),
        mesh=mesh,
        scratch_types=[
            pltpu.VMEM((_PW,), jnp.int32),
            pltpu.VMEM((_NCH, _CH, _D), jnp.float32),
            pltpu.SemaphoreType.DMA,
        ],
        compiler_params=pltpu.CompilerParams(needs_layout_passes=False),
    )
    def run(idx_hbm, orf_hbm, out_hbm, kv, rows, gsem):
        wid = lax.axis_index("s") * 2 + lax.axis_index("c")
        base = wid * _PW
        pltpu.sync_copy(idx_hbm.at[pl.ds(base, _PW)], kv)
        gcopies = []
        for c in range(_NCH):
            gcopies.append(pltpu.async_copy(
                orf_hbm.at[kv.at[pl.ds(c * _CH, _CH)]], rows.at[c], gsem))
        for c in range(_NCH):
            gcopies[c].wait()
            pltpu.sync_copy(rows.at[c],
                            out_hbm.at[pl.ds(base + c * _CH, _CH)])

    return run(slot_idx, orf0)


def _split_bf16(x):
    hi = x.astype(jnp.bfloat16)
    lo = (x - hi.astype(jnp.float32)).astype(jnp.bfloat16)
    return hi, lo


def _tc_body(num_ref, noff_ref, eoff_ref, node_hbm, edge_hbm, gn_hbm,
             es_v, ed_v, a1_ref, a2_ref, a12_ref, ow_ref,
             outf_ref, ps_ref, pd_ref,
             node_v, edge_v, gn_v, nsem, esem, gsem):
    g = pl.program_id(0)
    n = num_ref[g]
    no = noff_ref[g]
    eo = eoff_ref[g]
    n3 = n * 3
    ow0 = ow_ref[0:1, :]
    ow1 = ow_ref[1:2, :]

    cp_g = pltpu.make_async_copy(gn_hbm.at[pl.ds(no, _MAXN)], gn_v, gsem)
    cp_n = pltpu.make_async_copy(node_hbm.at[pl.ds(no, _MAXN)], node_v, nsem)
    cp_e = pltpu.make_async_copy(edge_hbm.at[pl.ds(eo, _MAXE)], edge_v, esem)
    cp_g.start()
    cp_n.start()
    cp_e.start()

    j = lax.broadcasted_iota(jnp.int32, (_MAXE, 1), 0)
    emask = j < n3
    t = lax.broadcasted_iota(jnp.int32, (_MAXLEN, 1), 0)
    base = jnp.where(t < n, t, 0)
    zi = jnp.zeros((), jnp.int32)
    ps_ref[...] = base
    pd_ref[...] = base
    ps_ref[pl.ds(n, _MAXE), :] = jnp.where(emask, es_v[...], zi)
    pd_ref[pl.ds(n, _MAXE), :] = jnp.where(emask, ed_v[...], zi)

    cp_g.wait()
    gn = gn_v[...]
    p1 = jnp.dot(gn, a1_ref[...], preferred_element_type=jnp.float32,
                 precision=lax.Precision.HIGHEST)
    p2 = jnp.dot(gn, a2_ref[...], preferred_element_type=jnp.float32,
                 precision=lax.Precision.HIGHEST)
    pn = jnp.dot(gn, a12_ref[...], preferred_element_type=jnp.float32,
                 precision=lax.Precision.HIGHEST)
    p1h, p1l = _split_bf16(p1)
    p2h, p2l = _split_bf16(p2)
    rhs = jnp.concatenate(
        [jnp.concatenate([p1h, p1l], axis=1),
         jnp.concatenate([p2h, p2l], axis=1)], axis=0)

    cp_n.wait()
    outf_ref[...] = jnp.zeros((_MAXLEN, _D), jnp.float32)
    outf_ref[0:_MAXN, :] = node_v[...] + pn + ow1
    cp_e.wait()

    tab16 = lax.broadcasted_iota(jnp.int16, (1, _MAXN), 1)
    oneb = jnp.ones((), jnp.bfloat16)
    zerob = jnp.zeros((), jnp.bfloat16)
    for r in range(_MAXE // _RTILE):
        lo = r * _RTILE
        es_r = es_v[lo:lo + _RTILE, :]
        ed_r = ed_v[lo:lo + _RTILE, :]
        ohs = jnp.where(es_r.astype(jnp.int16) == tab16, oneb, zerob)
        ohd = jnp.where(ed_r.astype(jnp.int16) == tab16, oneb, zerob)
        oh = jnp.concatenate([ohs, ohd], axis=1)
        eqf = (es_r == ed_r).astype(jnp.float32)
        owe = ow0 + eqf * (ow1 - ow0)
        res = jnp.dot(oh, rhs, preferred_element_type=jnp.float32)
        acc = res[:, :_D] + res[:, _D:]
        ce = edge_v[lo:lo + _RTILE, :] + acc + owe
        jr = lax.broadcasted_iota(jnp.int32, (_RTILE, 1), 0) + lo
        ce = jnp.where(jr < n3, ce, 0.0)
        outf_ref[pl.ds(n + lo, _RTILE), :] = ce


def _tc_call(num, noff, eoff, node_data, edge_data, gn,
             es_col, ed_col, a1, a2, a12, ow):
    def per_graph(*blk):
        return pl.BlockSpec(blk, lambda g: (g,) + (0,) * (len(blk) - 1))

    def shared(*blk):
        return pl.BlockSpec(blk, lambda g: (0,) * len(blk))

    smem = pl.BlockSpec(memory_space=pltpu.SMEM)
    hbm = pl.BlockSpec(memory_space=pl.ANY)
    return pl.pallas_call(
        _tc_body,
        grid=(_B,),
        in_specs=[
            smem, smem, smem,
            hbm, hbm, hbm,
            per_graph(None, _MAXE, 1),
            per_graph(None, _MAXE, 1),
            shared(_D, _D),
            shared(_D, _D),
            shared(_D, _D),
            shared(2, _D),
        ],
        out_specs=[
            pl.BlockSpec((_MAXLEN, _D), lambda g: (0, g)),
            pl.BlockSpec((None, _MAXLEN, 1), lambda g: (g, 0, 0)),
            pl.BlockSpec((None, _MAXLEN, 1), lambda g: (g, 0, 0)),
        ],
        out_shape=[
            jax.ShapeDtypeStruct((_MAXLEN, _B * _D), jnp.float32),
            jax.ShapeDtypeStruct((_B, _MAXLEN, 1), jnp.int32),
            jax.ShapeDtypeStruct((_B, _MAXLEN, 1), jnp.int32),
        ],
        scratch_shapes=[
            pltpu.VMEM((_MAXN, _D), jnp.float32),
            pltpu.VMEM((_MAXE, _D), jnp.float32),
            pltpu.VMEM((_MAXN, _D), jnp.float32),
            pltpu.SemaphoreType.DMA,
            pltpu.SemaphoreType.DMA,
            pltpu.SemaphoreType.DMA,
        ],
        compiler_params=pltpu.CompilerParams(
            dimension_semantics=("arbitrary",)),
    )(num, noff, eoff, node_data, edge_data, gn,
      es_col, ed_col, a1, a2, a12, ow)


def _static_setup():
    node_num = np.asarray(_NODE_NUM, np.int32)
    noff = np.concatenate([[0], np.cumsum(node_num)]).astype(np.int64)
    eoff = 3 * noff
    tok = np.arange(_MAXLEN, dtype=np.int32)[None, :]
    nn = node_num[:, None]
    seq = 4 * nn
    padded_node_mask = tok < nn
    padded_edge_mask = (tok >= nn) & (tok < seq)
    padding_mask = tok >= seq
    return noff, eoff, padding_mask, padded_node_mask, padded_edge_mask


_NOFF, _EOFF, _PAD_MASK, _NODE_MASK, _EDGE_MASK = _static_setup()


def kernel(node_data, edge_data, edge_index, indices_subnodes,
           order_weight, orf_w, orf):
    orf_flat = orf.reshape(10 * 50000, _D)
    es_cols, ed_cols = [], []
    for g, n in enumerate(_NODE_NUM):
        e = 3 * n
        es_cols.append(jnp.pad(edge_index[0, _EOFF[g]:_EOFF[g] + e],
                               (0, _MAXE - e)))
        ed_cols.append(jnp.pad(edge_index[1, _EOFF[g]:_EOFF[g] + e],
                               (0, _MAXE - e)))
    es_col = jnp.stack(es_cols).astype(jnp.int32)[..., None]
    ed_col = jnp.stack(ed_cols).astype(jnp.int32)[..., None]

    gn = _sc_gather(indices_subnodes.astype(jnp.int32), orf_flat)

    a1 = orf_w[:, :_D].T
    a2 = orf_w[:, _D:].T
    a12 = a1 + a2
    num = jnp.asarray(np.asarray(_NODE_NUM, np.int32))
    noff = jnp.asarray(_NOFF[:_B].astype(np.int32))
    eoff = jnp.asarray(_EOFF[:_B].astype(np.int32))

    outf, ps, pd = _tc_call(num, noff, eoff,
                            node_data, edge_data, gn,
                            es_col, ed_col, a1, a2, a12,
                            order_weight)

    padded_feature = jnp.transpose(
        outf.reshape(_MAXLEN, _B, _D), (1, 0, 2))
    padded_index = jnp.stack([ps[..., 0], pd[..., 0]], axis=-1)
    return (padded_feature,
            jnp.asarray(_PAD_MASK),
            padded_index,
            jnp.asarray(_NODE_MASK),
            jnp.asarray(_EDGE_MASK))
